# Initial kernel scaffold; baseline (speedup 1.0000x reference)
import jax
import jax.numpy as jnp
from jax.experimental import pallas as pl


def kernel(x, edge_index, edge_weight, edge_label_index, params):
    raise NotImplementedError("write your pallas kernel here")


# speedup vs baseline: 3.5138x; 3.5138x over previous
import functools

import jax
import jax.numpy as jnp
from jax import lax
from jax.experimental import pallas as pl
from jax.experimental.pallas import tpu as pltpu
from jax.experimental.pallas import tpu_sc as plsc

_N = 10000
_E = 320000
_EL = 100000
_EPS = 1e-5

_NTILES = 16
_CHUNK = 128
_NP = 10240


def _sc_mesh():
    return plsc.VectorSubcoreMesh(core_axis_name="c", subcore_axis_name="s")


def _spmm_sc(hA, hB, src, dst, wflat, zblock, split_dims):
    dh = hA.shape[1]
    e2p = src.shape[0]
    per_tile = e2p // _NTILES if split_dims else e2p // (2 * _NTILES)
    nchunks = per_tile // _CHUNK
    rows_per_tile = _NP // _NTILES

    @functools.partial(
        pl.kernel,
        mesh=_sc_mesh(),
        out_type=[
            jax.ShapeDtypeStruct((_NP, dh), jnp.float32),
            jax.ShapeDtypeStruct((_NP, dh), jnp.float32),
        ],
        scratch_types=[
            pltpu.VMEM((_CHUNK,), jnp.int32),
            pltpu.VMEM((_CHUNK,), jnp.int32),
            pltpu.VMEM((_CHUNK * 16,), jnp.float32),
            pltpu.VMEM((_CHUNK, dh), jnp.float32),
            pltpu.VMEM_SHARED((_NP, dh), jnp.float32),
            pltpu.SemaphoreType.DMA,
        ],
    )
    def k(hA_hbm, hB_hbm, src_hbm, dst_hbm, w_hbm, z_hbm, outA, outB,
          sidx_v, didx_v, w_v, rows_v, acc, sem):
        c = lax.axis_index("c")
        s = lax.axis_index("s")
        r0 = s * rows_per_tile
        pltpu.sync_copy(z_hbm.at[pl.ds(r0, rows_per_tile)],
                        acc.at[pl.ds(r0, rows_per_tile)])
        plsc.subcore_barrier()

        if split_dims:
            base = s * per_tile
        else:
            base = (c * _NTILES + s) * per_tile

        def body(j, carry):
            off = base + j * _CHUNK
            pltpu.sync_copy(src_hbm.at[pl.ds(off, _CHUNK)], sidx_v)
            pltpu.sync_copy(dst_hbm.at[pl.ds(off, _CHUNK)], didx_v)
            pltpu.sync_copy(w_hbm.at[pl.ds(off * 16, _CHUNK * 16)], w_v)

            @pl.when(c == 0)
            def _():
                pltpu.async_copy(hA_hbm.at[sidx_v], rows_v, sem).wait()

            @pl.when(c != 0)
            def _():
                pltpu.async_copy(hB_hbm.at[sidx_v], rows_v, sem).wait()

            def scale(e, carry2):
                wb = w_v[pl.ds(e * 16, 16)]
                for d in range(dh // 16):
                    rows_v[e, pl.ds(d * 16, 16)] = (
                        rows_v[e, pl.ds(d * 16, 16)] * wb)
                return carry2

            lax.fori_loop(0, _CHUNK, scale, 0)
            pltpu.sync_copy(rows_v, acc.at[didx_v], add=True)
            return carry

        lax.fori_loop(0, nchunks, body, 0)
        plsc.subcore_barrier()

        @pl.when(c == 0)
        def _():
            pltpu.sync_copy(acc.at[pl.ds(r0, rows_per_tile)],
                            outA.at[pl.ds(r0, rows_per_tile)])

        @pl.when(c != 0)
        def _():
            pltpu.sync_copy(acc.at[pl.ds(r0, rows_per_tile)],
                            outB.at[pl.ds(r0, rows_per_tile)])

    return k(hA, hB, src, dst, wflat, zblock)


def _deg_sum(degA, degB):
    blk = 1000

    def body(a_ref, b_ref, o_ref):
        o_ref[...] = a_ref[:, :16] + b_ref[:, :16]

    return pl.pallas_call(
        body,
        grid=(_N // blk,),
        in_specs=[pl.BlockSpec((blk, 128), lambda i: (i, 0))] * 2,
        out_specs=pl.BlockSpec((blk, 16), lambda i: (i, 0)),
        out_shape=jax.ShapeDtypeStruct((_N, 16), jnp.float32),
    )(degA[:_N], degB[:_N])


def _gather_pairs_sc(x, enc, eli0, eli1, el_pad):
    per_w = el_pad // 32
    nchunks = per_w // _CHUNK
    d = x.shape[1]

    @functools.partial(
        pl.kernel,
        mesh=_sc_mesh(),
        out_type=[jax.ShapeDtypeStruct((el_pad, d), jnp.float32)] * 4,
        scratch_types=[
            pltpu.VMEM((_CHUNK,), jnp.int32),
            pltpu.VMEM((_CHUNK, d), jnp.float32),
            pltpu.SemaphoreType.DMA,
        ],
    )
    def k(x_hbm, enc_hbm, a_hbm, b_hbm, f0, f1, f2, f3, idx_v, rows_v, sem):
        c = lax.axis_index("c")
        s = lax.axis_index("s")
        wid = s * 2 + c
        base = wid * per_w

        def body(j, carry):
            off = base + j * _CHUNK
            pltpu.sync_copy(a_hbm.at[pl.ds(off, _CHUNK)], idx_v)
            pltpu.async_copy(x_hbm.at[idx_v], rows_v, sem).wait()
            pltpu.sync_copy(rows_v, f0.at[pl.ds(off, _CHUNK)])
            pltpu.async_copy(enc_hbm.at[idx_v], rows_v, sem).wait()
            pltpu.sync_copy(rows_v, f1.at[pl.ds(off, _CHUNK)])
            pltpu.sync_copy(b_hbm.at[pl.ds(off, _CHUNK)], idx_v)
            pltpu.async_copy(x_hbm.at[idx_v], rows_v, sem).wait()
            pltpu.sync_copy(rows_v, f2.at[pl.ds(off, _CHUNK)])
            pltpu.async_copy(enc_hbm.at[idx_v], rows_v, sem).wait()
            pltpu.sync_copy(rows_v, f3.at[pl.ds(off, _CHUNK)])
            return carry

        lax.fori_loop(0, nchunks, body, 0)

    return k(x, enc, eli0, eli1)


def _dinv_of(deg_blk):
    return lax.rsqrt(jnp.clip(deg_blk[:, :1], 1e-12, None))


def _mm_deg(x, W, degm):
    n, din = x.shape
    dout = W.shape[1]
    blk = 1000

    def body(x_ref, w_ref, deg_ref, o_ref):
        o_ref[...] = _dinv_of(deg_ref[...]) * jnp.dot(
            x_ref[...], w_ref[...], preferred_element_type=jnp.float32)

    return pl.pallas_call(
        body,
        grid=(n // blk,),
        in_specs=[
            pl.BlockSpec((blk, din), lambda i: (i, 0)),
            pl.BlockSpec((din, dout), lambda i: (0, 0)),
            pl.BlockSpec((blk, 16), lambda i: (i, 0)),
        ],
        out_specs=pl.BlockSpec((blk, dout), lambda i: (i, 0)),
        out_shape=jax.ShapeDtypeStruct((n, dout), jnp.float32),
    )(x, W, degm)


def _post_stats(aggA, aggB, degm, b, combine):
    n, dh = aggA.shape
    d = 2 * dh if combine == 'concat' else dh
    blk = 1000
    g = n // blk

    def body(aA, aB, deg_ref, b_ref, o_ref, st_ref, acc):
        i = pl.program_id(0)

        @pl.when(i == 0)
        def _():
            acc[...] = jnp.zeros_like(acc)

        if combine == 'concat':
            agg = jnp.concatenate([aA[...], aB[...]], axis=1)
        else:
            agg = aA[...] + aB[...]
        out = agg * _dinv_of(deg_ref[...]) + b_ref[...]
        o_ref[...] = out
        acc[0:1, :] += jnp.sum(out, axis=0, keepdims=True)
        acc[1:2, :] += jnp.sum(out * out, axis=0, keepdims=True)

        @pl.when(i == g - 1)
        def _():
            st_ref[...] = acc[...]

    return pl.pallas_call(
        body,
        grid=(g,),
        in_specs=[
            pl.BlockSpec((blk, dh), lambda i: (i, 0)),
            pl.BlockSpec((blk, dh), lambda i: (i, 0)),
            pl.BlockSpec((blk, 16), lambda i: (i, 0)),
            pl.BlockSpec((1, d), lambda i: (0, 0)),
        ],
        out_specs=[
            pl.BlockSpec((blk, d), lambda i: (i, 0)),
            pl.BlockSpec((8, d), lambda i: (0, 0)),
        ],
        out_shape=[
            jax.ShapeDtypeStruct((n, d), jnp.float32),
            jax.ShapeDtypeStruct((8, d), jnp.float32),
        ],
        scratch_shapes=[pltpu.VMEM((8, d), jnp.float32)],
    )(aggA, aggB, degm, b.reshape(1, -1))


def _bn_relu_mm_deg(h, stats, gam, beta, W, degm):
    n, d = h.shape
    dout = W.shape[1]
    blk = 1000

    def body(h_ref, st_ref, g_ref, be_ref, w_ref, deg_ref, o_ref):
        m = st_ref[0:1, :] / n
        v = st_ref[1:2, :] / n - m * m
        rstd = lax.rsqrt(v + _EPS)
        xb = jnp.maximum((h_ref[...] - m) * (rstd * g_ref[...]) + be_ref[...],
                         0.0)
        o_ref[...] = _dinv_of(deg_ref[...]) * jnp.dot(
            xb, w_ref[...], preferred_element_type=jnp.float32)

    return pl.pallas_call(
        body,
        grid=(n // blk,),
        in_specs=[
            pl.BlockSpec((blk, d), lambda i: (i, 0)),
            pl.BlockSpec((8, d), lambda i: (0, 0)),
            pl.BlockSpec((1, d), lambda i: (0, 0)),
            pl.BlockSpec((1, d), lambda i: (0, 0)),
            pl.BlockSpec((d, dout), lambda i: (0, 0)),
            pl.BlockSpec((blk, 16), lambda i: (i, 0)),
        ],
        out_specs=pl.BlockSpec((blk, dout), lambda i: (i, 0)),
        out_shape=jax.ShapeDtypeStruct((n, dout), jnp.float32),
    )(h, stats, gam.reshape(1, -1), beta.reshape(1, -1), W, degm)


def _dec_first(f0, f1, f2, f3, W, b):
    elp, dh = f0.shape
    dout = W.shape[1]
    blk = 2048
    g = elp // blk

    def body(f0r, f1r, f2r, f3r, w_ref, b_ref, o_ref, st_ref, acc):
        i = pl.program_id(0)

        @pl.when(i == 0)
        def _():
            acc[...] = jnp.zeros_like(acc)

        h = (jnp.dot(f0r[...], w_ref[0:dh], preferred_element_type=jnp.float32)
             + jnp.dot(f1r[...], w_ref[dh:2 * dh],
                       preferred_element_type=jnp.float32)
             + jnp.dot(f2r[...], w_ref[2 * dh:3 * dh],
                       preferred_element_type=jnp.float32)
             + jnp.dot(f3r[...], w_ref[3 * dh:4 * dh],
                       preferred_element_type=jnp.float32)) + b_ref[...]
        o_ref[...] = h
        rid = i * blk + lax.broadcasted_iota(jnp.int32, (blk, 1), 0)
        hm = jnp.where(rid < _EL, h, 0.0)
        acc[0:1, :] += jnp.sum(hm, axis=0, keepdims=True)
        acc[1:2, :] += jnp.sum(hm * h, axis=0, keepdims=True)

        @pl.when(i == g - 1)
        def _():
            st_ref[...] = acc[...]

    return pl.pallas_call(
        body,
        grid=(g,),
        in_specs=[pl.BlockSpec((blk, dh), lambda i: (i, 0))] * 4 + [
            pl.BlockSpec((4 * dh, dout), lambda i: (0, 0)),
            pl.BlockSpec((1, dout), lambda i: (0, 0)),
        ],
        out_specs=[
            pl.BlockSpec((blk, dout), lambda i: (i, 0)),
            pl.BlockSpec((8, dout), lambda i: (0, 0)),
        ],
        out_shape=[
            jax.ShapeDtypeStruct((elp, dout), jnp.float32),
            jax.ShapeDtypeStruct((8, dout), jnp.float32),
        ],
        scratch_shapes=[pltpu.VMEM((8, dout), jnp.float32)],
    )(f0, f1, f2, f3, W, b.reshape(1, -1))


def _dec_mid(h, stats, gam, beta, W, b):
    elp, d = h.shape
    dout = W.shape[1]
    blk = 2048
    g = elp // blk

    def body(h_ref, st_ref, g_ref, be_ref, w_ref, b_ref, o_ref, st2_ref, acc):
        i = pl.program_id(0)

        @pl.when(i == 0)
        def _():
            acc[...] = jnp.zeros_like(acc)

        m = st_ref[0:1, :] / _EL
        v = st_ref[1:2, :] / _EL - m * m
        rstd = lax.rsqrt(v + _EPS)
        xb = jnp.maximum((h_ref[...] - m) * (rstd * g_ref[...]) + be_ref[...],
                         0.0)
        h2 = jnp.dot(xb, w_ref[...],
                     preferred_element_type=jnp.float32) + b_ref[...]
        o_ref[...] = h2
        rid = i * blk + lax.broadcasted_iota(jnp.int32, (blk, 1), 0)
        hm = jnp.where(rid < _EL, h2, 0.0)
        acc[0:1, :] += jnp.sum(hm, axis=0, keepdims=True)
        acc[1:2, :] += jnp.sum(hm * h2, axis=0, keepdims=True)

        @pl.when(i == g - 1)
        def _():
            st2_ref[...] = acc[...]

    return pl.pallas_call(
        body,
        grid=(g,),
        in_specs=[
            pl.BlockSpec((blk, d), lambda i: (i, 0)),
            pl.BlockSpec((8, d), lambda i: (0, 0)),
            pl.BlockSpec((1, d), lambda i: (0, 0)),
            pl.BlockSpec((1, d), lambda i: (0, 0)),
            pl.BlockSpec((d, dout), lambda i: (0, 0)),
            pl.BlockSpec((1, dout), lambda i: (0, 0)),
        ],
        out_specs=[
            pl.BlockSpec((blk, dout), lambda i: (i, 0)),
            pl.BlockSpec((8, dout), lambda i: (0, 0)),
        ],
        out_shape=[
            jax.ShapeDtypeStruct((elp, dout), jnp.float32),
            jax.ShapeDtypeStruct((8, dout), jnp.float32),
        ],
        scratch_shapes=[pltpu.VMEM((8, dout), jnp.float32)],
    )(h, stats, gam.reshape(1, -1), beta.reshape(1, -1), W, b.reshape(1, -1))


def _dec_final(h, stats, gam, beta, W, b):
    elp, d = h.shape
    blk = 2048

    def body(h_ref, st_ref, g_ref, be_ref, w_ref, b_ref, o_ref):
        m = st_ref[0:1, :] / _EL
        v = st_ref[1:2, :] / _EL - m * m
        rstd = lax.rsqrt(v + _EPS)
        xb = jnp.maximum((h_ref[...] - m) * (rstd * g_ref[...]) + be_ref[...],
                         0.0)
        z = jnp.dot(xb, w_ref[...],
                    preferred_element_type=jnp.float32) + b_ref[...]
        o_ref[...] = 1.0 / (1.0 + jnp.exp(-z))

    return pl.pallas_call(
        body,
        grid=(elp // blk,),
        in_specs=[
            pl.BlockSpec((blk, d), lambda i: (i, 0)),
            pl.BlockSpec((8, d), lambda i: (0, 0)),
            pl.BlockSpec((1, d), lambda i: (0, 0)),
            pl.BlockSpec((1, d), lambda i: (0, 0)),
            pl.BlockSpec((d, 1), lambda i: (0, 0)),
            pl.BlockSpec((1, 1), lambda i: (0, 0)),
        ],
        out_specs=pl.BlockSpec((blk, 1), lambda i: (i, 0)),
        out_shape=jax.ShapeDtypeStruct((elp, 1), jnp.float32),
    )(h, stats, gam.reshape(1, -1), beta.reshape(1, -1), W, b.reshape(1, -1))


def _predict_k(enc_stats, w1, b1, w2, b2, w3, b3):
    d = w1.shape[0]

    def body(st_ref, w1_ref, b1_ref, w2_ref, b2_ref, w3_ref, b3_ref, o_ref):
        hmean = st_ref[0:1, :] / _N
        h = jnp.maximum(
            jnp.dot(hmean, w1_ref[...], preferred_element_type=jnp.float32)
            + b1_ref[...], 0.0)
        h = jnp.maximum(
            jnp.dot(h, w2_ref[...], preferred_element_type=jnp.float32)
            + b2_ref[...], 0.0)
        logits = jnp.dot(h, w3_ref[...],
                         preferred_element_type=jnp.float32) + b3_ref[...]
        zs = logits - jnp.max(logits, axis=1, keepdims=True)
        ez = jnp.exp(zs)
        o_ref[...] = ez / jnp.sum(ez, axis=1, keepdims=True)

    return pl.pallas_call(
        body,
        in_specs=[
            pl.BlockSpec((8, d), lambda: (0, 0)),
            pl.BlockSpec((d, d), lambda: (0, 0)),
            pl.BlockSpec((1, d), lambda: (0, 0)),
            pl.BlockSpec((d, d), lambda: (0, 0)),
            pl.BlockSpec((1, d), lambda: (0, 0)),
            pl.BlockSpec((d, 3), lambda: (0, 0)),
            pl.BlockSpec((1, 3), lambda: (0, 0)),
        ],
        out_specs=pl.BlockSpec((1, 3), lambda: (0, 0)),
        out_shape=jax.ShapeDtypeStruct((1, 3), jnp.float32),
    )(enc_stats, w1, b1.reshape(1, -1), w2, b2.reshape(1, -1), w3,
      b3.reshape(1, -1))


def kernel(x, edge_index, edge_weight, edge_label_index, params):
    p = params
    x = x.astype(jnp.float32)

    e2 = _E + _N
    e2p = ((e2 + _NTILES * _CHUNK - 1) // (_NTILES * _CHUNK)) * (_NTILES * _CHUNK)
    loop = jnp.arange(_N, dtype=jnp.int32)
    padi = jnp.zeros((e2p - e2,), jnp.int32)
    src2 = jnp.concatenate([edge_index[0].astype(jnp.int32), loop, padi])
    dst2 = jnp.concatenate([edge_index[1].astype(jnp.int32), loop, padi])
    ew2 = jnp.concatenate([
        edge_weight.astype(jnp.float32),
        jnp.ones((_N,), jnp.float32),
        jnp.zeros((e2p - e2,), jnp.float32),
    ])
    wflat = jnp.broadcast_to(ew2[:, None], (e2p, 16)).reshape(-1)

    ones128 = jnp.ones((_N, 128), jnp.float32)
    z128 = jnp.zeros((_NP, 128), jnp.float32)
    degA, degB = _spmm_sc(ones128, ones128, src2, dst2, wflat, z128, False)
    degm = _deg_sum(degA, degB)

    hs = _mm_deg(x, p['W1'], degm)
    aggA, aggB = _spmm_sc(hs[:, :128], hs[:, 128:], src2, dst2, wflat, z128,
                          True)
    h1, st1 = _post_stats(aggA[:_N], aggB[:_N], degm, p['b1'], 'concat')
    hs = _bn_relu_mm_deg(h1, st1, p['g1'], p['be1'], p['W2'], degm)
    aggA, aggB = _spmm_sc(hs[:, :128], hs[:, 128:], src2, dst2, wflat, z128,
                          True)
    h2, st2 = _post_stats(aggA[:_N], aggB[:_N], degm, p['b2'], 'concat')
    hs = _bn_relu_mm_deg(h2, st2, p['g2'], p['be2'], p['W3'], degm)
    aggA, aggB = _spmm_sc(hs, hs, src2, dst2, wflat, z128, False)
    enc, st3 = _post_stats(aggA[:_N], aggB[:_N], degm, p['b3'], 'sum')

    k_pred = _predict_k(st3, p['Pw1'], p['Pb1'], p['Pw2'], p['Pb2'],
                        p['Pw3'], p['Pb3'])

    el_pad = ((_EL + 32 * _CHUNK - 1) // (32 * _CHUNK)) * (32 * _CHUNK)
    pade = jnp.zeros((el_pad - _EL,), jnp.int32)
    eli0 = jnp.concatenate([edge_label_index[0].astype(jnp.int32), pade])
    eli1 = jnp.concatenate([edge_label_index[1].astype(jnp.int32), pade])
    f0, f1, f2, f3 = _gather_pairs_sc(x, enc, eli0, eli1, el_pad)

    hd1, std1 = _dec_first(f0, f1, f2, f3, p['Dw1'], p['Db1'])
    hd2, std2 = _dec_mid(hd1, std1, p['Dg1'], p['Dbe1'], p['Dw2'], p['Db2'])
    decf = _dec_final(hd2, std2, p['Dg2'], p['Dbe2'], p['Dw3'], p['Db3'])
    dec = decf[:_EL, 0]

    return (dec, k_pred)
